# baseline (speedup 1.0000x reference)
import functools

import jax
import jax.numpy as jnp
from jax import lax
from jax.experimental import pallas as pl
from jax.experimental.pallas import tpu as pltpu
from jax.experimental.pallas import tpu_sc as plsc

_B = 4096
_D = 4096
_C = 20000
_E = 256
_M = 128
_NNZ = 163840

_CB = 1000
_NCB = _C // _CB

_NC = 2
_NS = 16
_NW = _NC * _NS
_NNZ_PER = _NNZ // _NW
_CHUNK = 128
_NCHUNK = _NNZ_PER // _CHUNK
_BROWS = _B // _NS
_ZROWS = 64


def _mm_body(ce_ref, w_ref, b_ref, mp_ref, out_ref):
    pce = jnp.dot(ce_ref[...], w_ref[...], preferred_element_type=jnp.float32)
    pce = pce + b_ref[...]
    pce = jax.nn.swish(pce)
    upd = lax.dot_general(mp_ref[...], pce, (((0,), (0,)), ((), ())),
                          preferred_element_type=jnp.float32)

    @pl.when(pl.program_id(0) == 0)
    def _():
        out_ref[...] = upd

    @pl.when(pl.program_id(0) != 0)
    def _():
        out_ref[...] += upd


def _matmul_pcm(card_embeddings, w, bias_row, metapath):
    return pl.pallas_call(
        _mm_body,
        grid=(_NCB,),
        in_specs=[
            pl.BlockSpec((_CB, _E), lambda i: (i, 0)),
            pl.BlockSpec((_E, _M), lambda i: (0, 0)),
            pl.BlockSpec((1, _M), lambda i: (0, 0)),
            pl.BlockSpec((_CB, _D), lambda i: (i, 0)),
        ],
        out_specs=pl.BlockSpec((_D, _M), lambda i: (0, 0)),
        out_shape=jax.ShapeDtypeStruct((_D, _M), jnp.float32),
        compiler_params=pltpu.CompilerParams(
            dimension_semantics=("arbitrary",),
        ),
    )(card_embeddings, w, bias_row, metapath)


def _sc_body(table_hbm, cols_hbm, rows_hbm, vals_hbm, out_hbm,
             cols_v, rows_v, vals_v, gb, acc,
             stsem, g0, g1, g2, g3, s0, s1, s2, s3):
    gsems = (g0, g1, g2, g3)
    ssems = (s0, s1, s2, s3)
    cid = lax.axis_index("c")
    sid = lax.axis_index("s")
    wid = cid * _NS + sid

    st0 = pltpu.async_copy(cols_hbm.at[wid], cols_v, stsem)
    st1 = pltpu.async_copy(rows_hbm.at[wid], rows_v, stsem)
    st2 = pltpu.async_copy(vals_hbm.at[wid], vals_v, stsem)

    zv = jnp.zeros((16,), jnp.float32)
    zbb = gb.at[3]

    def _zero_body(r, _):
        for k in range(_M // 16):
            zbb[r, pl.ds(k * 16, 16)] = zv
        return 0

    lax.fori_loop(0, _ZROWS, _zero_body, 0)
    st0.wait()
    st1.wait()
    st2.wait()

    def _fire(c, b):
        return pltpu.async_copy(table_hbm.at[cols_v.at[c]], gb.at[b],
                                gsems[b])

    _fire(0, 0)
    _fire(1, 1)

    zc = [pltpu.async_copy(gb.at[3, pl.ds(0, _ZROWS)],
                           acc.at[pl.ds(sid * _BROWS + q * _ZROWS, _ZROWS)],
                           stsem)
          for q in range(_BROWS // _ZROWS)]
    for d in zc:
        d.wait()
    plsc.subcore_barrier()

    def _chunk_quad(i, _):
        for j in range(4):
            c = 4 * i + j
            jn = (j + 2) % 4
            pltpu.make_async_copy(table_hbm.at[cols_v.at[c]], gb.at[j],
                                  gsems[j]).wait()

            base = c * _CHUNK
            gbb = gb.at[j]

            @plsc.parallel_loop(0, _CHUNK // 16, 1, unroll=4)
            def _blk(g):
                vg = vals_v[pl.ds(pl.multiple_of(base + g * 16, 16), 16)]
                for l in range(16):
                    v = vg.at[jnp.full((16,), l, jnp.int32)].get(
                        mode="promise_in_bounds")
                    r = g * 16 + l
                    for k in range(_M // 16):
                        gbb[r, pl.ds(k * 16, 16)] = gbb[r, pl.ds(k * 16, 16)] * v

            @pl.when(c >= 2)
            def _():
                cm2 = jnp.maximum(c - 2, 0)
                pltpu.make_async_copy(gb.at[jn], acc.at[rows_v.at[cm2]],
                                      ssems[jn]).wait()

            @pl.when(c + 2 < _NCHUNK)
            def _():
                _fire(c + 2, jn)

            pltpu.async_copy(gbb, acc.at[rows_v.at[c]], ssems[j], add=True)
        return 0

    lax.fori_loop(0, _NCHUNK // 4, _chunk_quad, 0)
    pltpu.make_async_copy(gb.at[(_NCHUNK - 2) % 4],
                          acc.at[rows_v.at[_NCHUNK - 2]],
                          ssems[(_NCHUNK - 2) % 4]).wait()
    pltpu.make_async_copy(gb.at[(_NCHUNK - 1) % 4],
                          acc.at[rows_v.at[_NCHUNK - 1]],
                          ssems[(_NCHUNK - 1) % 4]).wait()
    plsc.subcore_barrier()

    pltpu.sync_copy(acc.at[pl.ds(sid * _BROWS, _BROWS)],
                    out_hbm.at[cid, pl.ds(sid * _BROWS, _BROWS)])


_sc_call = functools.partial(
    pl.kernel,
    out_type=jax.ShapeDtypeStruct((_NC, _B, _M), jnp.float32),
    mesh=plsc.VectorSubcoreMesh(core_axis_name="c", subcore_axis_name="s"),
    scratch_types=[
        pltpu.VMEM((_NCHUNK, _CHUNK), jnp.int32),
        pltpu.VMEM((_NCHUNK, _CHUNK), jnp.int32),
        pltpu.VMEM((_NNZ_PER,), jnp.float32),
        pltpu.VMEM((4, _CHUNK, _M), jnp.float32),
        pltpu.VMEM_SHARED((_B, _M), jnp.float32),
        pltpu.SemaphoreType.DMA,
        pltpu.SemaphoreType.DMA,
        pltpu.SemaphoreType.DMA,
        pltpu.SemaphoreType.DMA,
        pltpu.SemaphoreType.DMA,
        pltpu.SemaphoreType.DMA,
        pltpu.SemaphoreType.DMA,
        pltpu.SemaphoreType.DMA,
        pltpu.SemaphoreType.DMA,
    ],
    compiler_params=pltpu.CompilerParams(use_tc_tiling_on_sc=False),
)(_sc_body)


def _add_body(p_ref, o_ref):
    o_ref[...] = p_ref[0] + p_ref[1]


def _combine(parts):
    nblk = 4
    return pl.pallas_call(
        _add_body,
        grid=(nblk,),
        in_specs=[pl.BlockSpec((_NC, _B // nblk, _M), lambda i: (0, i, 0))],
        out_specs=pl.BlockSpec((_B // nblk, _M), lambda i: (i, 0)),
        out_shape=jax.ShapeDtypeStruct((_B, _M), jnp.float32),
        compiler_params=pltpu.CompilerParams(
            dimension_semantics=("arbitrary",),
        ),
    )(parts)


def kernel(pool_values, card_embeddings, metapath, kernel, bias, pool_rows, pool_cols):
    bias_row = bias.reshape(1, _M)
    table = _matmul_pcm(card_embeddings, kernel, bias_row, metapath)
    cols = pool_cols.astype(jnp.int32).reshape(_NW, _NCHUNK, _CHUNK)
    rows = pool_rows.astype(jnp.int32).reshape(_NW, _NCHUNK, _CHUNK)
    vals = pool_values.reshape(_NW, _NNZ_PER)
    parts = _sc_call(table, cols, rows, vals)
    return _combine(parts)
